# Initial kernel scaffold; baseline (speedup 1.0000x reference)
import jax
import jax.numpy as jnp
from jax.experimental import pallas as pl


def kernel(x, edge_index, edge_label_index, W1, b1, W2, b2):
    raise NotImplementedError("write your pallas kernel here")


# speedup vs baseline: 5.0089x; 5.0089x over previous
import functools

import jax
import jax.numpy as jnp
from jax import lax
from jax.experimental import pallas as pl
from jax.experimental.pallas import tpu as pltpu
from jax.experimental.pallas import tpu_sc as plsc

NC = 2
NS = 16
LN = 16
NT = NC * NS
WIN = 128

f32 = jnp.float32
i32 = jnp.int32


@functools.lru_cache(maxsize=None)
def _make_deg(E, N):
    ept = E // NT
    nwin = ept // WIN
    tail = ept - nwin * WIN
    npad = ((N + NS * LN - 1) // (NS * LN)) * (NS * LN)
    sl = npad // NS
    assert E % NT == 0 and tail % 8 == 0 and sl % 16 == 0
    mesh = plsc.VectorSubcoreMesh(core_axis_name="c", subcore_axis_name="s")

    @functools.partial(
        pl.kernel, mesh=mesh,
        out_type=jax.ShapeDtypeStruct((NC * npad,), f32),
        scratch_types=[
            pltpu.VMEM((1, WIN), i32),
            pltpu.VMEM((1, 16), i32),
            pltpu.VMEM((WIN,), f32),
            pltpu.VMEM((sl,), f32),
            pltpu.VMEM_SHARED((npad,), f32),
        ],
    )
    def deg_kernel(dst_hbm, out_hbm, idx_v, idxt_v, ones_v, zbuf, deg_sh):
        c = lax.axis_index("c")
        s = lax.axis_index("s")
        wid = c * NS + s
        one16 = jnp.ones((LN,), f32)
        zero16 = jnp.zeros((LN,), f32)
        for i in range(WIN // LN):
            ones_v[pl.ds(i * LN, LN)] = one16
        for i in range(sl // LN):
            zbuf[pl.ds(i * LN, LN)] = zero16
        pltpu.sync_copy(zbuf, deg_sh.at[pl.ds(s * sl, sl)])
        plsc.subcore_barrier()
        e_base = wid * ept

        def body(w, carry):
            off = e_base + w * WIN
            pltpu.sync_copy(dst_hbm.at[pl.ds(off, WIN)], idx_v.at[0])
            pltpu.sync_copy(ones_v, deg_sh.at[idx_v.at[0]], add=True)
            return carry

        lax.fori_loop(0, nwin, body, 0)
        if tail:
            toff = e_base + nwin * WIN
            pltpu.sync_copy(dst_hbm.at[pl.ds(toff, tail)], idxt_v.at[0])
            pltpu.sync_copy(ones_v.at[pl.ds(0, tail)],
                            deg_sh.at[idxt_v.at[0]], add=True)
        plsc.subcore_barrier()
        pltpu.sync_copy(deg_sh.at[pl.ds(s * sl, sl)],
                        out_hbm.at[pl.ds(c * npad + s * sl, sl)])

    return deg_kernel, npad


@functools.lru_cache(maxsize=None)
def _make_scatter(E, N, D):
    ept = E // NT
    nwin = ept // WIN
    tail = ept - nwin * WIN
    rb = (N // NS) // 8 * 8
    tailr = N - NS * rb
    zr = 16
    assert rb % zr == 0 and tailr % 8 == 0 and tailr <= zr
    assert tail % 8 == 0 and D % LN == 0
    mesh = plsc.VectorSubcoreMesh(core_axis_name="c", subcore_axis_name="s")

    @functools.partial(
        pl.kernel, mesh=mesh,
        out_type=jax.ShapeDtypeStruct((NC * N, D), f32),
        scratch_types=[
            pltpu.VMEM((1, WIN), i32),
            pltpu.VMEM((1, WIN), i32),
            pltpu.VMEM((1, 16), i32),
            pltpu.VMEM((1, 16), i32),
            pltpu.VMEM((WIN, D), f32),
            pltpu.VMEM((zr, D), f32),
            pltpu.VMEM_SHARED((N, D), f32),
            pltpu.SemaphoreType.DMA,
        ],
    )
    def scatter_kernel(h_hbm, src_hbm, dst_hbm, out_hbm,
                       sidx, didx, sidx_t, didx_t, rows_v, zrow, acc_sh, sem):
        c = lax.axis_index("c")
        s = lax.axis_index("s")
        wid = c * NS + s
        zero16 = jnp.zeros((LN,), f32)
        for r in range(zr):
            for i in range(D // LN):
                zrow[r, pl.ds(i * LN, LN)] = zero16
        row_base = s * rb
        for i in range(rb // zr):
            pltpu.sync_copy(zrow, acc_sh.at[pl.ds(row_base + i * zr, zr)])
        if tailr:
            @pl.when(s == 0)
            def _():
                pltpu.sync_copy(zrow.at[pl.ds(0, tailr)],
                                acc_sh.at[pl.ds(NS * rb, tailr)])
        plsc.subcore_barrier()
        e_base = wid * ept

        def body(w, carry):
            off = e_base + w * WIN
            pltpu.sync_copy(src_hbm.at[pl.ds(off, WIN)], sidx.at[0])
            pltpu.sync_copy(dst_hbm.at[pl.ds(off, WIN)], didx.at[0])
            pltpu.async_copy(h_hbm.at[sidx.at[0]], rows_v, sem).wait()
            pltpu.sync_copy(rows_v, acc_sh.at[didx.at[0]], add=True)
            return carry

        lax.fori_loop(0, nwin, body, 0)
        if tail:
            toff = e_base + nwin * WIN
            pltpu.sync_copy(src_hbm.at[pl.ds(toff, tail)], sidx_t.at[0])
            pltpu.sync_copy(dst_hbm.at[pl.ds(toff, tail)], didx_t.at[0])
            pltpu.async_copy(h_hbm.at[sidx_t.at[0]],
                             rows_v.at[pl.ds(0, tail)], sem).wait()
            pltpu.sync_copy(rows_v.at[pl.ds(0, tail)],
                            acc_sh.at[didx_t.at[0]], add=True)
        plsc.subcore_barrier()
        pltpu.sync_copy(acc_sh.at[pl.ds(row_base, rb)],
                        out_hbm.at[pl.ds(c * N + row_base, rb)])
        if tailr:
            @pl.when(s == 0)
            def _():
                pltpu.sync_copy(acc_sh.at[pl.ds(NS * rb, tailr)],
                                out_hbm.at[pl.ds(c * N + NS * rb, tailr)])

    return scatter_kernel


@functools.lru_cache(maxsize=None)
def _make_decode(E, N, D):
    ept = E // NT
    nwin = ept // WIN
    tail = ept - nwin * WIN
    assert tail % 16 == 0 and D % LN == 0
    mesh = plsc.VectorSubcoreMesh(core_axis_name="c", subcore_axis_name="s")

    @functools.partial(
        pl.kernel, mesh=mesh,
        compiler_params=pltpu.CompilerParams(needs_layout_passes=False),
        out_type=jax.ShapeDtypeStruct((E,), f32),
        scratch_types=[
            pltpu.VMEM((1, WIN), i32),
            pltpu.VMEM((1, WIN), i32),
            pltpu.VMEM((WIN, D), f32),
            pltpu.VMEM((WIN, D), f32),
            pltpu.VMEM((WIN,), f32),
            pltpu.SemaphoreType.DMA,
            pltpu.SemaphoreType.DMA,
        ],
    )
    def decode_kernel(z_hbm, src_hbm, dst_hbm, out_hbm,
                      sidx, didx, rows_s, rows_d, obuf, sem_s, sem_d):
        c = lax.axis_index("c")
        s = lax.axis_index("s")
        wid = c * NS + s
        e_base = wid * ept
        lane = lax.iota(i32, LN)

        def do_window(off, ws):
            ng = ws // LN
            if ws == WIN:
                sidx_r, didx_r = sidx.at[0], didx.at[0]
                rs, rd = rows_s, rows_d
            else:
                sidx_r = sidx.at[0].at[pl.ds(0, ws)]
                didx_r = didx.at[0].at[pl.ds(0, ws)]
                rs = rows_s.at[pl.ds(0, ws)]
                rd = rows_d.at[pl.ds(0, ws)]
            pltpu.sync_copy(src_hbm.at[pl.ds(off, ws)], sidx_r)
            pltpu.sync_copy(dst_hbm.at[pl.ds(off, ws)], didx_r)
            cs = pltpu.async_copy(z_hbm.at[sidx_r], rs, sem_s)
            cd = pltpu.async_copy(z_hbm.at[didx_r], rd, sem_d)
            cs.wait()
            cd.wait()
            ids = [lane + g * LN for g in range(ng)]

            def kbody(k, accs):
                kvec = jnp.full((LN,), 0, i32) + k
                out = []
                for g in range(ng):
                    vs = plsc.load_gather(rows_s, [ids[g], kvec])
                    vd = plsc.load_gather(rows_d, [ids[g], kvec])
                    out.append(accs[g] + vs * vd)
                return tuple(out)

            accs = lax.fori_loop(0, D, kbody,
                                 tuple(jnp.zeros((LN,), f32) for _ in range(ng)))
            for g in range(ng):
                p = 1.0 / (1.0 + jnp.exp(-accs[g]))
                obuf[pl.ds(g * LN, LN)] = p
            pltpu.sync_copy(obuf.at[pl.ds(0, ws)], out_hbm.at[pl.ds(off, ws)])

        def body(w, carry):
            do_window(e_base + w * WIN, WIN)
            return carry

        lax.fori_loop(0, nwin, body, 0)
        if tail:
            do_window(e_base + nwin * WIN, tail)

    return decode_kernel


def _dinv(p0, p1):
    return lax.rsqrt(p0 + p1 + 1.0)


@functools.lru_cache(maxsize=None)
def _make_enc1(N, D, BN):
    def body(x_ref, w_ref, p0_ref, p1_ref, o_ref):
        dinv = _dinv(p0_ref[...], p1_ref[...])
        h = jnp.dot(x_ref[...], w_ref[...], preferred_element_type=f32)
        o_ref[...] = h * dinv

    grid = (N // BN,)
    return pl.pallas_call(
        body,
        grid=grid,
        in_specs=[
            pl.BlockSpec((BN, D), lambda j: (j, 0)),
            pl.BlockSpec((D, D), lambda j: (0, 0)),
            pl.BlockSpec((BN, 1), lambda j: (j, 0)),
            pl.BlockSpec((BN, 1), lambda j: (j, 0)),
        ],
        out_specs=pl.BlockSpec((BN, D), lambda j: (j, 0)),
        out_shape=jax.ShapeDtypeStruct((N, D), f32),
    )


@functools.lru_cache(maxsize=None)
def _make_enc2(N, D, BN):
    def body(a0_ref, a1_ref, hp_ref, p0_ref, p1_ref, b_ref, w_ref, o_ref):
        dinv = _dinv(p0_ref[...], p1_ref[...])
        pre = (hp_ref[...] + a0_ref[...] + a1_ref[...]) * dinv + b_ref[...]
        z = jnp.maximum(pre, 0.0)
        o_ref[...] = jnp.dot(z, w_ref[...], preferred_element_type=f32) * dinv

    grid = (N // BN,)
    return pl.pallas_call(
        body,
        grid=grid,
        in_specs=[
            pl.BlockSpec((BN, D), lambda j: (j, 0)),
            pl.BlockSpec((BN, D), lambda j: (j, 0)),
            pl.BlockSpec((BN, D), lambda j: (j, 0)),
            pl.BlockSpec((BN, 1), lambda j: (j, 0)),
            pl.BlockSpec((BN, 1), lambda j: (j, 0)),
            pl.BlockSpec((1, D), lambda j: (0, 0)),
            pl.BlockSpec((D, D), lambda j: (0, 0)),
        ],
        out_specs=pl.BlockSpec((BN, D), lambda j: (j, 0)),
        out_shape=jax.ShapeDtypeStruct((N, D), f32),
    )


@functools.lru_cache(maxsize=None)
def _make_final(N, D, BN):
    def body(a0_ref, a1_ref, hp_ref, p0_ref, p1_ref, b_ref, o_ref):
        dinv = _dinv(p0_ref[...], p1_ref[...])
        o_ref[...] = (hp_ref[...] + a0_ref[...] + a1_ref[...]) * dinv + b_ref[...]

    grid = (N // BN,)
    return pl.pallas_call(
        body,
        grid=grid,
        in_specs=[
            pl.BlockSpec((BN, D), lambda j: (j, 0)),
            pl.BlockSpec((BN, D), lambda j: (j, 0)),
            pl.BlockSpec((BN, D), lambda j: (j, 0)),
            pl.BlockSpec((BN, 1), lambda j: (j, 0)),
            pl.BlockSpec((BN, 1), lambda j: (j, 0)),
            pl.BlockSpec((1, D), lambda j: (0, 0)),
        ],
        out_specs=pl.BlockSpec((BN, D), lambda j: (j, 0)),
        out_shape=jax.ShapeDtypeStruct((N, D), f32),
    )


def kernel(x, edge_index, edge_label_index, W1, b1, W2, b2):
    N, D = x.shape
    E = edge_index.shape[1]
    EL = edge_label_index.shape[1]
    BN = 2000 if N % 2000 == 0 else 1250
    assert N % BN == 0

    src = edge_index[0].astype(i32)
    dst = edge_index[1].astype(i32)
    lsrc = edge_label_index[0].astype(i32)
    ldst = edge_label_index[1].astype(i32)
    x = x.astype(f32)

    deg_kernel, npad = _make_deg(E, N)
    degf = deg_kernel(dst)
    p0 = degf[0:N].reshape(N, 1)
    p1 = degf[npad:npad + N].reshape(N, 1)

    h1p = _make_enc1(N, D, BN)(x, W1, p0, p1)

    scatter = _make_scatter(E, N, D)
    acc1 = scatter(h1p, src, dst)
    h2p = _make_enc2(N, D, BN)(acc1[:N], acc1[N:], h1p, p0, p1,
                               b1.reshape(1, D), W2)
    acc2 = scatter(h2p, src, dst)
    z2 = _make_final(N, D, BN)(acc2[:N], acc2[N:], h2p, p0, p1,
                               b2.reshape(1, D))

    prob = _make_decode(EL, N, D)(z2, lsrc, ldst)
    return prob


# speedup vs baseline: 15.2229x; 3.0392x over previous
import functools

import jax
import jax.numpy as jnp
from jax import lax
from jax.experimental import pallas as pl
from jax.experimental.pallas import tpu as pltpu
from jax.experimental.pallas import tpu_sc as plsc

NC = 2
NS = 16
LN = 16
NT = NC * NS
WIN = 128

f32 = jnp.float32
i32 = jnp.int32


def _windows(E):
    qw = E // WIN
    assert qw * WIN == E
    maxw = ((qw + NT - 1) // NT + 7) // 8 * 8
    qwp = NT * maxw
    return qw, maxw, qwp


def _tile_windows(wid, qw, maxw):
    wstart = wid * maxw
    nw = jnp.clip(qw - wstart, 0, maxw)
    return nw, wstart


@functools.lru_cache(maxsize=None)
def _make_deg(E, N):
    qw, maxw, qwp = _windows(E)
    npad = ((N + NS * LN - 1) // (NS * LN)) * (NS * LN)
    sl = npad // NS
    lag = 4
    mesh = plsc.VectorSubcoreMesh(core_axis_name="c", subcore_axis_name="s")

    @functools.partial(
        pl.kernel, mesh=mesh,
        out_type=jax.ShapeDtypeStruct((NC * npad,), f32),
        scratch_types=[
            pltpu.VMEM((maxw, WIN), i32),
            pltpu.VMEM((WIN,), f32),
            pltpu.VMEM((sl,), f32),
            pltpu.VMEM_SHARED((npad,), f32),
            pltpu.SemaphoreType.DMA,
        ],
    )
    def deg_kernel(dst_hbm, out_hbm, didx, ones_v, zbuf, deg_sh, sem_s):
        c = lax.axis_index("c")
        s = lax.axis_index("s")
        wid = c * NS + s
        nw, wstart = _tile_windows(wid, qw, maxw)
        one16 = jnp.ones((LN,), f32)
        zero16 = jnp.zeros((LN,), f32)
        for i in range(WIN // LN):
            ones_v[pl.ds(i * LN, LN)] = one16
        for i in range(sl // LN):
            zbuf[pl.ds(i * LN, LN)] = zero16
        pltpu.sync_copy(zbuf, deg_sh.at[pl.ds(s * sl, sl)])
        pltpu.sync_copy(dst_hbm.at[pl.ds(wstart, maxw)], didx)
        plsc.subcore_barrier()

        def body(w, carry):
            pltpu.async_copy(ones_v, deg_sh.at[didx.at[w]], sem_s, add=True)

            @pl.when(w >= lag)
            def _():
                pltpu.make_async_copy(out_hbm.at[pl.ds(0, WIN)],
                                      ones_v, sem_s).wait()
            return carry

        lax.fori_loop(0, nw, body, 0)

        def drain(i, carry):
            pltpu.make_async_copy(out_hbm.at[pl.ds(0, WIN)],
                                  ones_v, sem_s).wait()
            return carry

        lax.fori_loop(0, jnp.minimum(nw, lag), drain, 0)
        plsc.subcore_barrier()
        pltpu.sync_copy(deg_sh.at[pl.ds(s * sl, sl)],
                        out_hbm.at[pl.ds(c * npad + s * sl, sl)])

    return deg_kernel, npad


@functools.lru_cache(maxsize=None)
def _make_scatter(E, N, D):
    qw, maxw, qwp = _windows(E)
    rb = (N // NS) // 8 * 8
    tailr = N - NS * rb
    zr = 16
    assert rb % zr == 0 and tailr % 8 == 0 and tailr <= zr and D % LN == 0
    mesh = plsc.VectorSubcoreMesh(core_axis_name="c", subcore_axis_name="s")

    ch = 8

    @functools.partial(
        pl.kernel, mesh=mesh,
        out_type=jax.ShapeDtypeStruct((NC * N, D), f32),
        scratch_types=[
            pltpu.VMEM((2, ch, WIN), i32),
            pltpu.VMEM((2, ch, WIN), i32),
            pltpu.VMEM((2 * WIN, D), f32),
            pltpu.VMEM((zr, D), f32),
            pltpu.VMEM_SHARED((N, D), f32),
            pltpu.SemaphoreType.DMA,
            pltpu.SemaphoreType.DMA,
        ],
    )
    def scatter_kernel(h_hbm, src_hbm, dst_hbm, out_hbm,
                       sidx, didx, rows2, zrow, acc_sh, sem_g, sem_s):
        c = lax.axis_index("c")
        s = lax.axis_index("s")
        wid = c * NS + s
        nw, wstart = _tile_windows(wid, qw, maxw)
        zero16 = jnp.zeros((LN,), f32)
        for r in range(zr):
            for i in range(D // LN):
                zrow[r, pl.ds(i * LN, LN)] = zero16
        row_base = s * rb
        for i in range(rb // zr):
            pltpu.sync_copy(zrow, acc_sh.at[pl.ds(row_base + i * zr, zr)])
        if tailr:
            @pl.when(s == 0)
            def _():
                pltpu.sync_copy(zrow.at[pl.ds(0, tailr)],
                                acc_sh.at[pl.ds(NS * rb, tailr)])
        plsc.subcore_barrier()

        def refill(w):
            par = (w // ch) % 2
            off = pl.multiple_of(wstart + w, 8)
            pltpu.sync_copy(src_hbm.at[pl.ds(off, ch)], sidx.at[par])
            pltpu.sync_copy(dst_hbm.at[pl.ds(off, ch)], didx.at[par])

        @pl.when(nw > 0)
        def _():
            refill(0)
            pltpu.async_copy(h_hbm.at[sidx.at[0, 0]],
                             rows2.at[pl.ds(0, WIN)], sem_g)

        def body(w, carry):
            cur = (w % 2) * WIN
            nxt = WIN - cur
            par = (w // ch) % 2
            pltpu.make_async_copy(h_hbm.at[pl.ds(0, WIN)],
                                  rows2.at[pl.ds(0, WIN)], sem_g).wait()

            @pl.when(w >= 1)
            def _():
                pltpu.make_async_copy(h_hbm.at[pl.ds(0, WIN)],
                                      rows2.at[pl.ds(0, WIN)], sem_s).wait()

            pltpu.async_copy(rows2.at[pl.ds(cur, WIN)],
                             acc_sh.at[didx.at[par, w % ch]], sem_s, add=True)

            @pl.when(w + 1 < nw)
            def _():
                @pl.when((w + 1) % ch == 0)
                def _():
                    refill(w + 1)
                npar = ((w + 1) // ch) % 2
                pltpu.async_copy(h_hbm.at[sidx.at[npar, (w + 1) % ch]],
                                 rows2.at[pl.ds(nxt, WIN)], sem_g)
            return carry

        lax.fori_loop(0, nw, body, 0)

        @pl.when(nw > 0)
        def _():
            pltpu.make_async_copy(h_hbm.at[pl.ds(0, WIN)],
                                  rows2.at[pl.ds(0, WIN)], sem_s).wait()
        plsc.subcore_barrier()
        pltpu.sync_copy(acc_sh.at[pl.ds(row_base, rb)],
                        out_hbm.at[pl.ds(c * N + row_base, rb)])
        if tailr:
            @pl.when(s == 0)
            def _():
                pltpu.sync_copy(acc_sh.at[pl.ds(NS * rb, tailr)],
                                out_hbm.at[pl.ds(c * N + NS * rb, tailr)])

    return scatter_kernel


@functools.lru_cache(maxsize=None)
def _make_decode(E, N, D):
    qw, maxw, qwp = _windows(E)
    assert D % LN == 0
    mesh = plsc.VectorSubcoreMesh(core_axis_name="c", subcore_axis_name="s")

    @functools.partial(
        pl.kernel, mesh=mesh,
        compiler_params=pltpu.CompilerParams(needs_layout_passes=False),
        out_type=jax.ShapeDtypeStruct((qw, WIN), f32),
        scratch_types=[
            pltpu.VMEM((maxw, WIN), i32),
            pltpu.VMEM((maxw, WIN), i32),
            pltpu.VMEM((2 * WIN, D), f32),
            pltpu.VMEM((2 * WIN, D), f32),
            pltpu.VMEM((2 * WIN,), f32),
            pltpu.SemaphoreType.DMA,
            pltpu.SemaphoreType.DMA,
        ],
    )
    def decode_kernel(z_hbm, src_hbm, dst_hbm, out_hbm,
                      sidx, didx, rows_s, rows_d, obuf, sem_g, sem_o):
        c = lax.axis_index("c")
        s = lax.axis_index("s")
        wid = c * NS + s
        nw, wstart = _tile_windows(wid, qw, maxw)
        lane = lax.iota(i32, LN)
        chunk_idx = [lane + ch * LN for ch in range(D // LN)]
        zero_f = jnp.zeros((LN,), f32)

        pltpu.sync_copy(src_hbm.at[pl.ds(wstart, maxw)], sidx)
        pltpu.sync_copy(dst_hbm.at[pl.ds(wstart, maxw)], didx)

        @pl.when(nw > 0)
        def _():
            pltpu.async_copy(z_hbm.at[sidx.at[0]],
                             rows_s.at[pl.ds(0, WIN)], sem_g)
            pltpu.async_copy(z_hbm.at[didx.at[0]],
                             rows_d.at[pl.ds(0, WIN)], sem_g)

        def body(w, carry):
            cur = (w % 2) * WIN
            nxt = WIN - cur
            for _ in range(2):
                pltpu.make_async_copy(z_hbm.at[pl.ds(0, WIN)],
                                      rows_s.at[pl.ds(0, WIN)], sem_g).wait()

            @pl.when(w >= 1)
            def _():
                pltpu.make_async_copy(out_hbm.at[0],
                                      obuf.at[pl.ds(0, WIN)], sem_o).wait()

            @pl.when(w + 1 < nw)
            def _():
                pltpu.async_copy(z_hbm.at[sidx.at[w + 1]],
                                 rows_s.at[pl.ds(nxt, WIN)], sem_g)
                pltpu.async_copy(z_hbm.at[didx.at[w + 1]],
                                 rows_d.at[pl.ds(nxt, WIN)], sem_g)

            def gbody(g, gcarry):
                ebase = cur + g * LN
                dots = zero_f
                for j in range(LN):
                    evec = jnp.zeros((LN,), i32) + (ebase + j)
                    acc = zero_f
                    for ch in range(D // LN):
                        vs = plsc.load_gather(rows_s, [evec, chunk_idx[ch]])
                        vd = plsc.load_gather(rows_d, [evec, chunk_idx[ch]])
                        acc = acc + vs * vd
                    dots = jnp.where(lane == j, jnp.sum(acc), dots)
                p = 1.0 / (1.0 + jnp.exp(-dots))
                plsc.store_scatter(obuf, [ebase + lane], p)
                return gcarry

            lax.fori_loop(0, WIN // LN, gbody, 0)
            pltpu.async_copy(obuf.at[pl.ds(cur, WIN)],
                             out_hbm.at[wstart + w], sem_o)
            return carry

        lax.fori_loop(0, nw, body, 0)

        @pl.when(nw > 0)
        def _():
            pltpu.make_async_copy(out_hbm.at[0],
                                  obuf.at[pl.ds(0, WIN)], sem_o).wait()

    return decode_kernel


def _dinv(p0, p1):
    return lax.rsqrt(p0 + p1 + 1.0)


@functools.lru_cache(maxsize=None)
def _make_enc1(N, D, BN):
    def body(x_ref, w_ref, p0_ref, p1_ref, o_ref):
        dinv = _dinv(p0_ref[...], p1_ref[...])
        h = jnp.dot(x_ref[...], w_ref[...], preferred_element_type=f32)
        o_ref[...] = h * dinv

    grid = (N // BN,)
    return pl.pallas_call(
        body,
        grid=grid,
        in_specs=[
            pl.BlockSpec((BN, D), lambda j: (j, 0)),
            pl.BlockSpec((D, D), lambda j: (0, 0)),
            pl.BlockSpec((BN, 1), lambda j: (j, 0)),
            pl.BlockSpec((BN, 1), lambda j: (j, 0)),
        ],
        out_specs=pl.BlockSpec((BN, D), lambda j: (j, 0)),
        out_shape=jax.ShapeDtypeStruct((N, D), f32),
    )


@functools.lru_cache(maxsize=None)
def _make_enc2(N, D, BN):
    def body(a0_ref, a1_ref, hp_ref, p0_ref, p1_ref, b_ref, w_ref, o_ref):
        dinv = _dinv(p0_ref[...], p1_ref[...])
        pre = (hp_ref[...] + a0_ref[...] + a1_ref[...]) * dinv + b_ref[...]
        z = jnp.maximum(pre, 0.0)
        o_ref[...] = jnp.dot(z, w_ref[...], preferred_element_type=f32) * dinv

    grid = (N // BN,)
    return pl.pallas_call(
        body,
        grid=grid,
        in_specs=[
            pl.BlockSpec((BN, D), lambda j: (j, 0)),
            pl.BlockSpec((BN, D), lambda j: (j, 0)),
            pl.BlockSpec((BN, D), lambda j: (j, 0)),
            pl.BlockSpec((BN, 1), lambda j: (j, 0)),
            pl.BlockSpec((BN, 1), lambda j: (j, 0)),
            pl.BlockSpec((1, D), lambda j: (0, 0)),
            pl.BlockSpec((D, D), lambda j: (0, 0)),
        ],
        out_specs=pl.BlockSpec((BN, D), lambda j: (j, 0)),
        out_shape=jax.ShapeDtypeStruct((N, D), f32),
    )


@functools.lru_cache(maxsize=None)
def _make_final(N, D, BN):
    def body(a0_ref, a1_ref, hp_ref, p0_ref, p1_ref, b_ref, o_ref):
        dinv = _dinv(p0_ref[...], p1_ref[...])
        o_ref[...] = (hp_ref[...] + a0_ref[...] + a1_ref[...]) * dinv + b_ref[...]

    grid = (N // BN,)
    return pl.pallas_call(
        body,
        grid=grid,
        in_specs=[
            pl.BlockSpec((BN, D), lambda j: (j, 0)),
            pl.BlockSpec((BN, D), lambda j: (j, 0)),
            pl.BlockSpec((BN, D), lambda j: (j, 0)),
            pl.BlockSpec((BN, 1), lambda j: (j, 0)),
            pl.BlockSpec((BN, 1), lambda j: (j, 0)),
            pl.BlockSpec((1, D), lambda j: (0, 0)),
        ],
        out_specs=pl.BlockSpec((BN, D), lambda j: (j, 0)),
        out_shape=jax.ShapeDtypeStruct((N, D), f32),
    )


def _prep_idx(a, E):
    qw, maxw, qwp = _windows(E)
    a2 = a.reshape(qw, WIN)
    if qwp > qw:
        a2 = jnp.concatenate([a2, jnp.zeros((qwp - qw, WIN), i32)], axis=0)
    return a2


def kernel(x, edge_index, edge_label_index, W1, b1, W2, b2):
    N, D = x.shape
    E = edge_index.shape[1]
    EL = edge_label_index.shape[1]
    BN = 2000 if N % 2000 == 0 else 1250
    assert N % BN == 0

    src = _prep_idx(edge_index[0].astype(i32), E)
    dst = _prep_idx(edge_index[1].astype(i32), E)
    lsrc = _prep_idx(edge_label_index[0].astype(i32), EL)
    ldst = _prep_idx(edge_label_index[1].astype(i32), EL)
    x = x.astype(f32)

    deg_kernel, npad = _make_deg(E, N)
    degf = deg_kernel(dst)
    p0 = degf[0:N].reshape(N, 1)
    p1 = degf[npad:npad + N].reshape(N, 1)

    h1p = _make_enc1(N, D, BN)(x, W1, p0, p1)

    scatter = _make_scatter(E, N, D)
    acc1 = scatter(h1p, src, dst)
    h2p = _make_enc2(N, D, BN)(acc1[:N], acc1[N:], h1p, p0, p1,
                               b1.reshape(1, D), W2)
    acc2 = scatter(h2p, src, dst)
    z2 = _make_final(N, D, BN)(acc2[:N], acc2[N:], h2p, p0, p1,
                               b2.reshape(1, D))

    prob = _make_decode(EL, N, D)(z2, lsrc, ldst)
    return prob.reshape(EL)


# speedup vs baseline: 16.4771x; 1.0824x over previous
import functools

import jax
import jax.numpy as jnp
from jax import lax
from jax.experimental import pallas as pl
from jax.experimental.pallas import tpu as pltpu
from jax.experimental.pallas import tpu_sc as plsc

NC = 2
NS = 16
LN = 16
NT = NC * NS
WIN = 128

f32 = jnp.float32
i32 = jnp.int32


def _windows(E):
    qw = E // WIN
    assert qw * WIN == E
    maxw = ((qw + NT - 1) // NT + 7) // 8 * 8
    qwp = NT * maxw
    return qw, maxw, qwp


def _tile_windows(wid, qw, maxw):
    wstart = wid * maxw
    nw = jnp.clip(qw - wstart, 0, maxw)
    return nw, wstart


@functools.lru_cache(maxsize=None)
def _make_deg(E, N):
    qw, maxw, qwp = _windows(E)
    npad = ((N + NS * LN - 1) // (NS * LN)) * (NS * LN)
    sl = npad // NS
    lag = 4
    mesh = plsc.VectorSubcoreMesh(core_axis_name="c", subcore_axis_name="s")

    @functools.partial(
        pl.kernel, mesh=mesh,
        out_type=jax.ShapeDtypeStruct((NC * npad,), f32),
        scratch_types=[
            pltpu.VMEM((maxw, WIN), i32),
            pltpu.VMEM((WIN,), f32),
            pltpu.VMEM((sl,), f32),
            pltpu.VMEM_SHARED((npad,), f32),
            pltpu.SemaphoreType.DMA,
        ],
    )
    def deg_kernel(dst_hbm, out_hbm, didx, ones_v, zbuf, deg_sh, sem_s):
        c = lax.axis_index("c")
        s = lax.axis_index("s")
        wid = c * NS + s
        nw, wstart = _tile_windows(wid, qw, maxw)
        one16 = jnp.ones((LN,), f32)
        zero16 = jnp.zeros((LN,), f32)
        for i in range(WIN // LN):
            ones_v[pl.ds(i * LN, LN)] = one16
        for i in range(sl // LN):
            zbuf[pl.ds(i * LN, LN)] = zero16
        pltpu.sync_copy(zbuf, deg_sh.at[pl.ds(s * sl, sl)])
        pltpu.sync_copy(dst_hbm.at[pl.ds(wstart, maxw)], didx)
        plsc.subcore_barrier()

        def body(w, carry):
            pltpu.async_copy(ones_v, deg_sh.at[didx.at[w]], sem_s, add=True)

            @pl.when(w >= lag)
            def _():
                pltpu.make_async_copy(out_hbm.at[pl.ds(0, WIN)],
                                      ones_v, sem_s).wait()
            return carry

        lax.fori_loop(0, nw, body, 0)

        def drain(i, carry):
            pltpu.make_async_copy(out_hbm.at[pl.ds(0, WIN)],
                                  ones_v, sem_s).wait()
            return carry

        lax.fori_loop(0, jnp.minimum(nw, lag), drain, 0)
        plsc.subcore_barrier()
        pltpu.sync_copy(deg_sh.at[pl.ds(s * sl, sl)],
                        out_hbm.at[pl.ds(c * npad + s * sl, sl)])

    return deg_kernel, npad


@functools.lru_cache(maxsize=None)
def _make_scatter(E, N, D):
    qw, maxw, qwp = _windows(E)
    rb = (N // NS) // 8 * 8
    tailr = N - NS * rb
    zr = 16
    assert rb % zr == 0 and tailr % 8 == 0 and tailr <= zr and D % LN == 0
    mesh = plsc.VectorSubcoreMesh(core_axis_name="c", subcore_axis_name="s")

    ch = 8

    @functools.partial(
        pl.kernel, mesh=mesh,
        out_type=jax.ShapeDtypeStruct((NC * N, D), f32),
        scratch_types=[
            pltpu.VMEM((2, ch, WIN), i32),
            pltpu.VMEM((2, ch, WIN), i32),
            pltpu.VMEM((2 * WIN, D), f32),
            pltpu.VMEM((zr, D), f32),
            pltpu.VMEM_SHARED((N, D), f32),
            pltpu.SemaphoreType.DMA,
            pltpu.SemaphoreType.DMA,
        ],
    )
    def scatter_kernel(h_hbm, src_hbm, dst_hbm, out_hbm,
                       sidx, didx, rows2, zrow, acc_sh, sem_g, sem_s):
        c = lax.axis_index("c")
        s = lax.axis_index("s")
        wid = c * NS + s
        nw, wstart = _tile_windows(wid, qw, maxw)
        zero16 = jnp.zeros((LN,), f32)
        for r in range(zr):
            for i in range(D // LN):
                zrow[r, pl.ds(i * LN, LN)] = zero16
        row_base = s * rb
        for i in range(rb // zr):
            pltpu.sync_copy(zrow, acc_sh.at[pl.ds(row_base + i * zr, zr)])
        if tailr:
            @pl.when(s == 0)
            def _():
                pltpu.sync_copy(zrow.at[pl.ds(0, tailr)],
                                acc_sh.at[pl.ds(NS * rb, tailr)])
        plsc.subcore_barrier()

        def refill(w):
            par = (w // ch) % 2
            off = pl.multiple_of(wstart + w, 8)
            pltpu.sync_copy(src_hbm.at[pl.ds(off, ch)], sidx.at[par])
            pltpu.sync_copy(dst_hbm.at[pl.ds(off, ch)], didx.at[par])

        @pl.when(nw > 0)
        def _():
            refill(0)
            pltpu.async_copy(h_hbm.at[sidx.at[0, 0]],
                             rows2.at[pl.ds(0, WIN)], sem_g)

        def body(w, carry):
            cur = (w % 2) * WIN
            nxt = WIN - cur
            par = (w // ch) % 2
            pltpu.make_async_copy(h_hbm.at[pl.ds(0, WIN)],
                                  rows2.at[pl.ds(0, WIN)], sem_g).wait()

            @pl.when(w >= 1)
            def _():
                pltpu.make_async_copy(h_hbm.at[pl.ds(0, WIN)],
                                      rows2.at[pl.ds(0, WIN)], sem_s).wait()

            pltpu.async_copy(rows2.at[pl.ds(cur, WIN)],
                             acc_sh.at[didx.at[par, w % ch]], sem_s, add=True)

            @pl.when(w + 1 < nw)
            def _():
                @pl.when((w + 1) % ch == 0)
                def _():
                    refill(w + 1)
                npar = ((w + 1) // ch) % 2
                pltpu.async_copy(h_hbm.at[sidx.at[npar, (w + 1) % ch]],
                                 rows2.at[pl.ds(nxt, WIN)], sem_g)
            return carry

        lax.fori_loop(0, nw, body, 0)

        @pl.when(nw > 0)
        def _():
            pltpu.make_async_copy(h_hbm.at[pl.ds(0, WIN)],
                                  rows2.at[pl.ds(0, WIN)], sem_s).wait()
        plsc.subcore_barrier()
        pltpu.sync_copy(acc_sh.at[pl.ds(row_base, rb)],
                        out_hbm.at[pl.ds(c * N + row_base, rb)])
        if tailr:
            @pl.when(s == 0)
            def _():
                pltpu.sync_copy(acc_sh.at[pl.ds(NS * rb, tailr)],
                                out_hbm.at[pl.ds(c * N + NS * rb, tailr)])

    return scatter_kernel


@functools.lru_cache(maxsize=None)
def _make_decode(E, N, D):
    qw, maxw, qwp = _windows(E)
    assert D % LN == 0
    mesh = plsc.VectorSubcoreMesh(core_axis_name="c", subcore_axis_name="s")

    @functools.partial(
        pl.kernel, mesh=mesh,
        compiler_params=pltpu.CompilerParams(needs_layout_passes=False),
        out_type=jax.ShapeDtypeStruct((qw, WIN), f32),
        scratch_types=[
            pltpu.VMEM((maxw, WIN), i32),
            pltpu.VMEM((maxw, WIN), i32),
            pltpu.VMEM((2 * WIN, D), f32),
            pltpu.VMEM((2 * WIN, D), f32),
            pltpu.VMEM((2 * WIN,), f32),
            pltpu.SemaphoreType.DMA,
            pltpu.SemaphoreType.DMA,
        ],
    )
    def decode_kernel(z_hbm, src_hbm, dst_hbm, out_hbm,
                      sidx, didx, rows_s, rows_d, obuf, sem_g, sem_o):
        c = lax.axis_index("c")
        s = lax.axis_index("s")
        wid = c * NS + s
        nw, wstart = _tile_windows(wid, qw, maxw)
        lane = lax.iota(i32, LN)
        chunk_idx = [lane + ch * LN for ch in range(D // LN)]
        zero_f = jnp.zeros((LN,), f32)

        pltpu.sync_copy(src_hbm.at[pl.ds(wstart, maxw)], sidx)
        pltpu.sync_copy(dst_hbm.at[pl.ds(wstart, maxw)], didx)

        @pl.when(nw > 0)
        def _():
            pltpu.async_copy(z_hbm.at[sidx.at[0]],
                             rows_s.at[pl.ds(0, WIN)], sem_g)
            pltpu.async_copy(z_hbm.at[didx.at[0]],
                             rows_d.at[pl.ds(0, WIN)], sem_g)

        def body(w, carry):
            cur = (w % 2) * WIN
            nxt = WIN - cur
            for _ in range(2):
                pltpu.make_async_copy(z_hbm.at[pl.ds(0, WIN)],
                                      rows_s.at[pl.ds(0, WIN)], sem_g).wait()

            @pl.when(w >= 1)
            def _():
                pltpu.make_async_copy(out_hbm.at[0],
                                      obuf.at[pl.ds(0, WIN)], sem_o).wait()

            @pl.when(w + 1 < nw)
            def _():
                pltpu.async_copy(z_hbm.at[sidx.at[w + 1]],
                                 rows_s.at[pl.ds(nxt, WIN)], sem_g)
                pltpu.async_copy(z_hbm.at[didx.at[w + 1]],
                                 rows_d.at[pl.ds(nxt, WIN)], sem_g)

            def gbody(g, gcarry):
                ebase = cur + g * LN
                dots = zero_f
                for j in range(LN):
                    e = ebase + j
                    acc = rows_s[e, pl.ds(0, LN)] * rows_d[e, pl.ds(0, LN)]
                    for ch in range(1, D // LN):
                        acc = acc + (rows_s[e, pl.ds(ch * LN, LN)] *
                                     rows_d[e, pl.ds(ch * LN, LN)])
                    dots = jnp.where(lane == j, jnp.sum(acc), dots)
                p = 1.0 / (1.0 + jnp.exp(-dots))
                plsc.store_scatter(obuf, [ebase + lane], p)
                return gcarry

            lax.fori_loop(0, WIN // LN, gbody, 0)
            pltpu.async_copy(obuf.at[pl.ds(cur, WIN)],
                             out_hbm.at[wstart + w], sem_o)
            return carry

        lax.fori_loop(0, nw, body, 0)

        @pl.when(nw > 0)
        def _():
            pltpu.make_async_copy(out_hbm.at[0],
                                  obuf.at[pl.ds(0, WIN)], sem_o).wait()

    return decode_kernel


def _dinv(p0, p1):
    return lax.rsqrt(p0 + p1 + 1.0)


@functools.lru_cache(maxsize=None)
def _make_enc1(N, D, BN):
    def body(x_ref, w_ref, p0_ref, p1_ref, o_ref):
        dinv = _dinv(p0_ref[...], p1_ref[...])
        h = jnp.dot(x_ref[...], w_ref[...], preferred_element_type=f32)
        o_ref[...] = h * dinv

    grid = (N // BN,)
    return pl.pallas_call(
        body,
        grid=grid,
        in_specs=[
            pl.BlockSpec((BN, D), lambda j: (j, 0)),
            pl.BlockSpec((D, D), lambda j: (0, 0)),
            pl.BlockSpec((BN, 1), lambda j: (j, 0)),
            pl.BlockSpec((BN, 1), lambda j: (j, 0)),
        ],
        out_specs=pl.BlockSpec((BN, D), lambda j: (j, 0)),
        out_shape=jax.ShapeDtypeStruct((N, D), f32),
    )


@functools.lru_cache(maxsize=None)
def _make_enc2(N, D, BN):
    def body(a0_ref, a1_ref, hp_ref, p0_ref, p1_ref, b_ref, w_ref, o_ref):
        dinv = _dinv(p0_ref[...], p1_ref[...])
        pre = (hp_ref[...] + a0_ref[...] + a1_ref[...]) * dinv + b_ref[...]
        z = jnp.maximum(pre, 0.0)
        o_ref[...] = jnp.dot(z, w_ref[...], preferred_element_type=f32) * dinv

    grid = (N // BN,)
    return pl.pallas_call(
        body,
        grid=grid,
        in_specs=[
            pl.BlockSpec((BN, D), lambda j: (j, 0)),
            pl.BlockSpec((BN, D), lambda j: (j, 0)),
            pl.BlockSpec((BN, D), lambda j: (j, 0)),
            pl.BlockSpec((BN, 1), lambda j: (j, 0)),
            pl.BlockSpec((BN, 1), lambda j: (j, 0)),
            pl.BlockSpec((1, D), lambda j: (0, 0)),
            pl.BlockSpec((D, D), lambda j: (0, 0)),
        ],
        out_specs=pl.BlockSpec((BN, D), lambda j: (j, 0)),
        out_shape=jax.ShapeDtypeStruct((N, D), f32),
    )


@functools.lru_cache(maxsize=None)
def _make_final(N, D, BN):
    def body(a0_ref, a1_ref, hp_ref, p0_ref, p1_ref, b_ref, o_ref):
        dinv = _dinv(p0_ref[...], p1_ref[...])
        o_ref[...] = (hp_ref[...] + a0_ref[...] + a1_ref[...]) * dinv + b_ref[...]

    grid = (N // BN,)
    return pl.pallas_call(
        body,
        grid=grid,
        in_specs=[
            pl.BlockSpec((BN, D), lambda j: (j, 0)),
            pl.BlockSpec((BN, D), lambda j: (j, 0)),
            pl.BlockSpec((BN, D), lambda j: (j, 0)),
            pl.BlockSpec((BN, 1), lambda j: (j, 0)),
            pl.BlockSpec((BN, 1), lambda j: (j, 0)),
            pl.BlockSpec((1, D), lambda j: (0, 0)),
        ],
        out_specs=pl.BlockSpec((BN, D), lambda j: (j, 0)),
        out_shape=jax.ShapeDtypeStruct((N, D), f32),
    )


def _prep_idx(a, E):
    qw, maxw, qwp = _windows(E)
    a2 = a.reshape(qw, WIN)
    if qwp > qw:
        a2 = jnp.concatenate([a2, jnp.zeros((qwp - qw, WIN), i32)], axis=0)
    return a2


def kernel(x, edge_index, edge_label_index, W1, b1, W2, b2):
    N, D = x.shape
    E = edge_index.shape[1]
    EL = edge_label_index.shape[1]
    BN = 2000 if N % 2000 == 0 else 1250
    assert N % BN == 0

    src = _prep_idx(edge_index[0].astype(i32), E)
    dst = _prep_idx(edge_index[1].astype(i32), E)
    lsrc = _prep_idx(edge_label_index[0].astype(i32), EL)
    ldst = _prep_idx(edge_label_index[1].astype(i32), EL)
    x = x.astype(f32)

    deg_kernel, npad = _make_deg(E, N)
    degf = deg_kernel(dst)
    p0 = degf[0:N].reshape(N, 1)
    p1 = degf[npad:npad + N].reshape(N, 1)

    h1p = _make_enc1(N, D, BN)(x, W1, p0, p1)

    scatter = _make_scatter(E, N, D)
    acc1 = scatter(h1p, src, dst)
    h2p = _make_enc2(N, D, BN)(acc1[:N], acc1[N:], h1p, p0, p1,
                               b1.reshape(1, D), W2)
    acc2 = scatter(h2p, src, dst)
    z2 = _make_final(N, D, BN)(acc2[:N], acc2[N:], h2p, p0, p1,
                               b2.reshape(1, D))

    prob = _make_decode(EL, N, D)(z2, lsrc, ldst)
    return prob.reshape(EL)


# speedup vs baseline: 21.1074x; 1.2810x over previous
import functools

import jax
import jax.numpy as jnp
from jax import lax
from jax.experimental import pallas as pl
from jax.experimental.pallas import tpu as pltpu
from jax.experimental.pallas import tpu_sc as plsc

NC = 2
NS = 16
LN = 16
NT = NC * NS
WIN = 128

f32 = jnp.float32
i32 = jnp.int32


def _windows(E, W):
    qw = E // W
    assert qw * W == E
    maxw = ((qw + NT - 1) // NT + 7) // 8 * 8
    qwp = NT * maxw
    return qw, maxw, qwp


def _tile_windows(wid, qw, maxw):
    wstart = wid * maxw
    nw = jnp.clip(qw - wstart, 0, maxw)
    return nw, wstart


@functools.lru_cache(maxsize=None)
def _make_deg(E, N):
    W = 64
    qw, maxw, qwp = _windows(E, W)
    npad = ((N + NS * LN - 1) // (NS * LN)) * (NS * LN)
    sl = npad // NS
    lag = 4
    mesh = plsc.VectorSubcoreMesh(core_axis_name="c", subcore_axis_name="s")

    @functools.partial(
        pl.kernel, mesh=mesh,
        out_type=jax.ShapeDtypeStruct((NC * npad,), f32),
        scratch_types=[
            pltpu.VMEM((maxw, W), i32),
            pltpu.VMEM((W,), f32),
            pltpu.VMEM((sl,), f32),
            pltpu.VMEM_SHARED((npad,), f32),
            pltpu.SemaphoreType.DMA,
        ],
    )
    def deg_kernel(dst_hbm, out_hbm, didx, ones_v, zbuf, deg_sh, sem_s):
        c = lax.axis_index("c")
        s = lax.axis_index("s")
        wid = c * NS + s
        nw, wstart = _tile_windows(wid, qw, maxw)
        one16 = jnp.ones((LN,), f32)
        zero16 = jnp.zeros((LN,), f32)
        for i in range(W // LN):
            ones_v[pl.ds(i * LN, LN)] = one16
        for i in range(sl // LN):
            zbuf[pl.ds(i * LN, LN)] = zero16
        pltpu.sync_copy(zbuf, deg_sh.at[pl.ds(s * sl, sl)])
        pltpu.sync_copy(dst_hbm.at[pl.ds(wstart, maxw)], didx)
        plsc.subcore_barrier()

        def body(w, carry):
            pltpu.async_copy(ones_v, deg_sh.at[didx.at[w]], sem_s, add=True)

            @pl.when(w >= lag)
            def _():
                pltpu.make_async_copy(out_hbm.at[pl.ds(0, W)],
                                      ones_v, sem_s).wait()
            return carry

        lax.fori_loop(0, nw, body, 0)

        def drain(i, carry):
            pltpu.make_async_copy(out_hbm.at[pl.ds(0, W)],
                                  ones_v, sem_s).wait()
            return carry

        lax.fori_loop(0, jnp.minimum(nw, lag), drain, 0)
        plsc.subcore_barrier()
        pltpu.sync_copy(deg_sh.at[pl.ds(s * sl, sl)],
                        out_hbm.at[pl.ds(c * npad + s * sl, sl)])

    return deg_kernel, npad


@functools.lru_cache(maxsize=None)
def _make_scatter(E, N, D):
    W = 64
    qw, maxw, qwp = _windows(E, W)
    rb = (N // NS) // 8 * 8
    tailr = N - NS * rb
    zr = 16
    assert rb % zr == 0 and tailr % 8 == 0 and tailr <= zr and D % LN == 0
    mesh = plsc.VectorSubcoreMesh(core_axis_name="c", subcore_axis_name="s")

    ch = 8

    @functools.partial(
        pl.kernel, mesh=mesh,
        out_type=jax.ShapeDtypeStruct((NC * N, D), f32),
        scratch_types=[
            pltpu.VMEM((2, ch, W), i32),
            pltpu.VMEM((2, ch, W), i32),
            pltpu.VMEM((4 * W, D), f32),
            pltpu.VMEM((zr, D), f32),
            pltpu.VMEM_SHARED((N, D), f32),
            pltpu.SemaphoreType.DMA,
            pltpu.SemaphoreType.DMA,
        ],
    )
    def scatter_kernel(h_hbm, src_hbm, dst_hbm, out_hbm,
                       sidx, didx, rows2, zrow, acc_sh, sem_g, sem_s):
        c = lax.axis_index("c")
        s = lax.axis_index("s")
        wid = c * NS + s
        nw, wstart = _tile_windows(wid, qw, maxw)
        zero16 = jnp.zeros((LN,), f32)
        for r in range(zr):
            for i in range(D // LN):
                zrow[r, pl.ds(i * LN, LN)] = zero16
        row_base = s * rb
        for i in range(rb // zr):
            pltpu.sync_copy(zrow, acc_sh.at[pl.ds(row_base + i * zr, zr)])
        if tailr:
            @pl.when(s == 0)
            def _():
                pltpu.sync_copy(zrow.at[pl.ds(0, tailr)],
                                acc_sh.at[pl.ds(NS * rb, tailr)])
        plsc.subcore_barrier()

        def refill(w):
            par = (w // ch) % 2
            off = pl.multiple_of(wstart + w, 8)
            pltpu.sync_copy(src_hbm.at[pl.ds(off, ch)], sidx.at[par])
            pltpu.sync_copy(dst_hbm.at[pl.ds(off, ch)], didx.at[par])

        @pl.when(nw > 0)
        def _():
            refill(0)
            pltpu.async_copy(h_hbm.at[sidx.at[0, 0]],
                             rows2.at[pl.ds(0, W)], sem_g)

        @pl.when(nw > 1)
        def _():
            pltpu.async_copy(h_hbm.at[sidx.at[0, 1]],
                             rows2.at[pl.ds(W, W)], sem_g)

        def body(w, carry):
            cur = (w % 4) * W
            par = (w // ch) % 2
            pltpu.make_async_copy(h_hbm.at[pl.ds(0, W)],
                                  rows2.at[pl.ds(0, W)], sem_g).wait()

            @pl.when(w >= 2)
            def _():
                pltpu.make_async_copy(h_hbm.at[pl.ds(0, W)],
                                      rows2.at[pl.ds(0, W)], sem_s).wait()

            pltpu.async_copy(rows2.at[pl.ds(cur, W)],
                             acc_sh.at[didx.at[par, w % ch]], sem_s, add=True)

            @pl.when(w + 2 < nw)
            def _():
                @pl.when((w + 2) % ch == 0)
                def _():
                    refill(w + 2)
                npar = ((w + 2) // ch) % 2
                nb = ((w + 2) % 4) * W
                pltpu.async_copy(h_hbm.at[sidx.at[npar, (w + 2) % ch]],
                                 rows2.at[pl.ds(nb, W)], sem_g)
            return carry

        lax.fori_loop(0, nw, body, 0)

        @pl.when(nw > 0)
        def _():
            pltpu.make_async_copy(h_hbm.at[pl.ds(0, W)],
                                  rows2.at[pl.ds(0, W)], sem_s).wait()

        @pl.when(nw > 1)
        def _():
            pltpu.make_async_copy(h_hbm.at[pl.ds(0, W)],
                                  rows2.at[pl.ds(0, W)], sem_s).wait()
        plsc.subcore_barrier()
        pltpu.sync_copy(acc_sh.at[pl.ds(row_base, rb)],
                        out_hbm.at[pl.ds(c * N + row_base, rb)])
        if tailr:
            @pl.when(s == 0)
            def _():
                pltpu.sync_copy(acc_sh.at[pl.ds(NS * rb, tailr)],
                                out_hbm.at[pl.ds(c * N + NS * rb, tailr)])

    return scatter_kernel


@functools.lru_cache(maxsize=None)
def _make_decode(E, N, D):
    qw, maxw, qwp = _windows(E, WIN)
    assert D % LN == 0
    pst = LN + 1
    mesh = plsc.VectorSubcoreMesh(core_axis_name="c", subcore_axis_name="s")

    @functools.partial(
        pl.kernel, mesh=mesh,
        compiler_params=pltpu.CompilerParams(needs_layout_passes=False),
        out_type=jax.ShapeDtypeStruct((qw, WIN), f32),
        scratch_types=[
            pltpu.VMEM((maxw, WIN), i32),
            pltpu.VMEM((maxw, WIN), i32),
            pltpu.VMEM((2 * WIN, D), f32),
            pltpu.VMEM((2 * WIN, D), f32),
            pltpu.VMEM((2 * WIN,), f32),
            pltpu.VMEM((LN * (LN + 1),), f32),
            pltpu.SemaphoreType.DMA,
            pltpu.SemaphoreType.DMA,
        ],
    )
    def decode_kernel(z_hbm, src_hbm, dst_hbm, out_hbm,
                      sidx, didx, rows_s, rows_d, obuf, pbuf, sem_g, sem_o):
        c = lax.axis_index("c")
        s = lax.axis_index("s")
        wid = c * NS + s
        nw, wstart = _tile_windows(wid, qw, maxw)
        lane = lax.iota(i32, LN)
        lane_pst = lane * pst

        pltpu.sync_copy(src_hbm.at[pl.ds(wstart, maxw)], sidx)
        pltpu.sync_copy(dst_hbm.at[pl.ds(wstart, maxw)], didx)

        @pl.when(nw > 0)
        def _():
            pltpu.async_copy(z_hbm.at[sidx.at[0]],
                             rows_s.at[pl.ds(0, WIN)], sem_g)
            pltpu.async_copy(z_hbm.at[didx.at[0]],
                             rows_d.at[pl.ds(0, WIN)], sem_g)

        def body(w, carry):
            cur = (w % 2) * WIN
            nxt = WIN - cur
            for _ in range(2):
                pltpu.make_async_copy(z_hbm.at[pl.ds(0, WIN)],
                                      rows_s.at[pl.ds(0, WIN)], sem_g).wait()

            @pl.when(w >= 1)
            def _():
                pltpu.make_async_copy(out_hbm.at[0],
                                      obuf.at[pl.ds(0, WIN)], sem_o).wait()

            @pl.when(w + 1 < nw)
            def _():
                pltpu.async_copy(z_hbm.at[sidx.at[w + 1]],
                                 rows_s.at[pl.ds(nxt, WIN)], sem_g)
                pltpu.async_copy(z_hbm.at[didx.at[w + 1]],
                                 rows_d.at[pl.ds(nxt, WIN)], sem_g)

            def gbody(g, gcarry):
                ebase = cur + g * LN
                for j in range(LN):
                    e = ebase + j
                    acc = rows_s[e, pl.ds(0, LN)] * rows_d[e, pl.ds(0, LN)]
                    for ch in range(1, D // LN):
                        acc = acc + (rows_s[e, pl.ds(ch * LN, LN)] *
                                     rows_d[e, pl.ds(ch * LN, LN)])
                    pbuf[pl.ds(j * pst, LN)] = acc
                dots = plsc.load_gather(pbuf, [lane_pst])
                for l in range(1, LN):
                    dots = dots + plsc.load_gather(pbuf, [lane_pst + l])
                p = 1.0 / (1.0 + jnp.exp(-dots))
                plsc.store_scatter(obuf, [ebase + lane], p)
                return gcarry

            lax.fori_loop(0, WIN // LN, gbody, 0)
            pltpu.async_copy(obuf.at[pl.ds(cur, WIN)],
                             out_hbm.at[wstart + w], sem_o)
            return carry

        lax.fori_loop(0, nw, body, 0)

        @pl.when(nw > 0)
        def _():
            pltpu.make_async_copy(out_hbm.at[0],
                                  obuf.at[pl.ds(0, WIN)], sem_o).wait()

    return decode_kernel


def _dinv(p0, p1):
    return lax.rsqrt(p0 + p1 + 1.0)


@functools.lru_cache(maxsize=None)
def _make_enc1(N, D, BN):
    def body(x_ref, w_ref, p0_ref, p1_ref, o_ref):
        dinv = _dinv(p0_ref[...], p1_ref[...])
        h = jnp.dot(x_ref[...], w_ref[...], preferred_element_type=f32)
        o_ref[...] = h * dinv

    grid = (N // BN,)
    return pl.pallas_call(
        body,
        grid=grid,
        in_specs=[
            pl.BlockSpec((BN, D), lambda j: (j, 0)),
            pl.BlockSpec((D, D), lambda j: (0, 0)),
            pl.BlockSpec((BN, 1), lambda j: (j, 0)),
            pl.BlockSpec((BN, 1), lambda j: (j, 0)),
        ],
        out_specs=pl.BlockSpec((BN, D), lambda j: (j, 0)),
        out_shape=jax.ShapeDtypeStruct((N, D), f32),
    )


@functools.lru_cache(maxsize=None)
def _make_enc2(N, D, BN):
    def body(a0_ref, a1_ref, hp_ref, p0_ref, p1_ref, b_ref, w_ref, o_ref):
        dinv = _dinv(p0_ref[...], p1_ref[...])
        pre = (hp_ref[...] + a0_ref[...] + a1_ref[...]) * dinv + b_ref[...]
        z = jnp.maximum(pre, 0.0)
        o_ref[...] = jnp.dot(z, w_ref[...], preferred_element_type=f32) * dinv

    grid = (N // BN,)
    return pl.pallas_call(
        body,
        grid=grid,
        in_specs=[
            pl.BlockSpec((BN, D), lambda j: (j, 0)),
            pl.BlockSpec((BN, D), lambda j: (j, 0)),
            pl.BlockSpec((BN, D), lambda j: (j, 0)),
            pl.BlockSpec((BN, 1), lambda j: (j, 0)),
            pl.BlockSpec((BN, 1), lambda j: (j, 0)),
            pl.BlockSpec((1, D), lambda j: (0, 0)),
            pl.BlockSpec((D, D), lambda j: (0, 0)),
        ],
        out_specs=pl.BlockSpec((BN, D), lambda j: (j, 0)),
        out_shape=jax.ShapeDtypeStruct((N, D), f32),
    )


@functools.lru_cache(maxsize=None)
def _make_final(N, D, BN):
    def body(a0_ref, a1_ref, hp_ref, p0_ref, p1_ref, b_ref, o_ref):
        dinv = _dinv(p0_ref[...], p1_ref[...])
        o_ref[...] = (hp_ref[...] + a0_ref[...] + a1_ref[...]) * dinv + b_ref[...]

    grid = (N // BN,)
    return pl.pallas_call(
        body,
        grid=grid,
        in_specs=[
            pl.BlockSpec((BN, D), lambda j: (j, 0)),
            pl.BlockSpec((BN, D), lambda j: (j, 0)),
            pl.BlockSpec((BN, D), lambda j: (j, 0)),
            pl.BlockSpec((BN, 1), lambda j: (j, 0)),
            pl.BlockSpec((BN, 1), lambda j: (j, 0)),
            pl.BlockSpec((1, D), lambda j: (0, 0)),
        ],
        out_specs=pl.BlockSpec((BN, D), lambda j: (j, 0)),
        out_shape=jax.ShapeDtypeStruct((N, D), f32),
    )


def _prep_idx(a, E, W):
    qw, maxw, qwp = _windows(E, W)
    a2 = a.reshape(qw, W)
    if qwp > qw:
        a2 = jnp.concatenate([a2, jnp.zeros((qwp - qw, W), i32)], axis=0)
    return a2


def kernel(x, edge_index, edge_label_index, W1, b1, W2, b2):
    N, D = x.shape
    E = edge_index.shape[1]
    EL = edge_label_index.shape[1]
    BN = 2000 if N % 2000 == 0 else 1250
    assert N % BN == 0

    src = _prep_idx(edge_index[0].astype(i32), E, 64)
    dst = _prep_idx(edge_index[1].astype(i32), E, 64)
    lsrc = _prep_idx(edge_label_index[0].astype(i32), EL, WIN)
    ldst = _prep_idx(edge_label_index[1].astype(i32), EL, WIN)
    x = x.astype(f32)

    deg_kernel, npad = _make_deg(E, N)
    degf = deg_kernel(dst)
    p0 = degf[0:N].reshape(N, 1)
    p1 = degf[npad:npad + N].reshape(N, 1)

    h1p = _make_enc1(N, D, BN)(x, W1, p0, p1)

    scatter = _make_scatter(E, N, D)
    acc1 = scatter(h1p, src, dst)
    h2p = _make_enc2(N, D, BN)(acc1[:N], acc1[N:], h1p, p0, p1,
                               b1.reshape(1, D), W2)
    acc2 = scatter(h2p, src, dst)
    z2 = _make_final(N, D, BN)(acc2[:N], acc2[N:], h2p, p0, p1,
                               b2.reshape(1, D))

    prob = _make_decode(EL, N, D)(z2, lsrc, ldst)
    return prob.reshape(EL)
